# baseline (speedup 1.0000x reference)
import functools

import jax
import jax.numpy as jnp
from jax import lax
from jax.experimental import pallas as pl
from jax.experimental.pallas import tpu as pltpu
from jax.experimental.pallas import tpu_sc as plsc

VOCAB = 100000
HIST_LEN = 200
BATCH = 16384
PADDING_IDX = 0

NUM_CORES = 2
NUM_SUBCORES = 16
LANES = 16
NW = NUM_CORES * NUM_SUBCORES
ROWS_PER_W = BATCH // NW
CHUNK = 128
NCHUNK = ROWS_PER_W // CHUNK

_WIN_OFFS = tuple(16 * j for j in range(12)) + (HIST_LEN - LANES,)

@functools.cache
def _build_history_view():
    mesh = plsc.VectorSubcoreMesh(core_axis_name="c", subcore_axis_name="s")

    @functools.partial(
        pl.kernel,
        out_type=(
            jax.ShapeDtypeStruct((BATCH, HIST_LEN), jnp.int32),
            jax.ShapeDtypeStruct((BATCH, HIST_LEN), jnp.int32),
        ),
        mesh=mesh,
        compiler_params=pltpu.CompilerParams(
            use_tc_tiling_on_sc=False,
            needs_layout_passes=False,
        ),
        scratch_types=[
            pltpu.VMEM((CHUNK,), jnp.int32),
            pltpu.VMEM((CHUNK, LANES), jnp.int32),
            pltpu.VMEM((CHUNK, HIST_LEN), jnp.int32),
            pltpu.VMEM((CHUNK, HIST_LEN), jnp.int32),
            pltpu.SemaphoreType.DMA,
        ],
    )
    def _history_view(hist_hbm, anchor_hbm, tgtb_hbm, padded_hbm, mask_hbm,
                      idx_v, tgt_v, rows_v, mask_v, sem):
        wid = lax.axis_index("s") * NUM_CORES + lax.axis_index("c")

        def row_body(r, _):
            tgt = tgt_v[r, pl.ds(0, LANES)]
            for off in _WIN_OFFS:
                h = rows_v[r, pl.ds(off, LANES)]
                keep = ~((h == tgt) | (h == PADDING_IDX))
                rows_v[r, pl.ds(off, LANES)] = jnp.where(keep, h, PADDING_IDX)
                mask_v[r, pl.ds(off, LANES)] = keep.astype(jnp.int32)
            return 0

        for c in range(NCHUNK):
            base = wid * ROWS_PER_W + c * CHUNK
            pltpu.sync_copy(anchor_hbm.at[pl.ds(base, CHUNK)], idx_v)
            pltpu.sync_copy(tgtb_hbm.at[pl.ds(base, CHUNK)], tgt_v)
            pltpu.async_copy(hist_hbm.at[idx_v], rows_v, sem).wait()
            lax.fori_loop(0, CHUNK, row_body, 0)
            pltpu.sync_copy(rows_v, padded_hbm.at[pl.ds(base, CHUNK)])
            pltpu.sync_copy(mask_v, mask_hbm.at[pl.ds(base, CHUNK)])

    return _history_view


def kernel(histories, anchor_idx, target_idx):
    out_dtype = histories.dtype
    tgt_bcast = jnp.broadcast_to(
        target_idx.astype(jnp.int32)[:, None], (BATCH, LANES))
    padded, mask_i32 = _build_history_view()(
        histories.astype(jnp.int32),
        anchor_idx.astype(jnp.int32),
        tgt_bcast,
    )
    return padded.astype(out_dtype), mask_i32.astype(jnp.bool_)
